# baseline (speedup 1.0000x reference)
import dataclasses
import functools

import jax
import jax.numpy as jnp
from jax import lax
from jax.experimental import pallas as pl
from jax.experimental.pallas import tpu as pltpu
from jax.experimental.pallas import tpu_sc as plsc

_N_IN = 50000
_N_OUT = 100000
_C = 128
_EPS = 1e-5
_SLOPE = 0.01
_BN = 10000


def _stats_body(h_ref, g_ref, b_ref, o_ref, acc_ref, *, nrows, nblocks):
    i = pl.program_id(0)

    @pl.when(i == 0)
    def _():
        acc_ref[...] = jnp.zeros_like(acc_ref)

    h = h_ref[...]
    h = jnp.where(h >= 0, h, _SLOPE * h)
    acc_ref[0, :] += jnp.sum(h, axis=0)
    acc_ref[1, :] += jnp.sum(h * h, axis=0)

    @pl.when(i == nblocks - 1)
    def _():
        m = acc_ref[0, :] / nrows
        v = acc_ref[1, :] / nrows - m * m
        a = g_ref[0, :] * jax.lax.rsqrt(v + _EPS)
        o_ref[0, :] = a
        o_ref[1, :] = b_ref[0, :] - m * a


def _lrelu_bn_stats(h, g, b, n):
    nblocks = n // _BN
    return pl.pallas_call(
        functools.partial(_stats_body, nrows=float(n), nblocks=nblocks),
        grid=(nblocks,),
        in_specs=[
            pl.BlockSpec((_BN, _C), lambda i: (i, 0)),
            pl.BlockSpec((1, _C), lambda i: (0, 0)),
            pl.BlockSpec((1, _C), lambda i: (0, 0)),
        ],
        out_specs=pl.BlockSpec((2, _C), lambda i: (0, 0)),
        out_shape=jax.ShapeDtypeStruct((2, _C), jnp.float32),
        scratch_shapes=[pltpu.VMEM((2, _C), jnp.float32)],
    )(h, g.reshape(1, _C), b.reshape(1, _C))


def _gemm_body(feat_ref, w_ref, ad_ref, y_ref, *, normed):
    k = pl.program_id(1)
    f = feat_ref[...]
    wk = w_ref[k]
    hi = jax.lax.Precision.DEFAULT
    if normed:
        a = ad_ref[0, :]
        d = ad_ref[1, :]
        f = jnp.where(f >= 0, f, _SLOPE * f)
        wk = a[:, None] * wk
        t = jnp.dot(d[None, :], w_ref[k], precision=hi,
                    preferred_element_type=jnp.float32)
        y_ref[0] = jnp.dot(f, wk, precision=hi,
                           preferred_element_type=jnp.float32) + t
    else:
        y_ref[0] = jnp.dot(f, wk, precision=hi,
                           preferred_element_type=jnp.float32)


def _dense_gemm(feat, W, ad, normed, n):
    K = W.shape[0]
    nblocks = n // _BN
    return pl.pallas_call(
        functools.partial(_gemm_body, normed=normed),
        grid=(nblocks, K),
        in_specs=[
            pl.BlockSpec((_BN, _C), lambda i, k: (i, 0)),
            pl.BlockSpec((K, _C, _C), lambda i, k: (0, 0, 0)),
            pl.BlockSpec((2, _C), lambda i, k: (0, 0)),
        ],
        out_specs=pl.BlockSpec((1, _BN, _C), lambda i, k: (k, i, 0)),
        out_shape=jax.ShapeDtypeStruct((K, n, _C), jnp.float32),
    )(feat, W, ad)


def _apply_body(h_ref, ad_ref, o_ref):
    h = h_ref[...]
    h = jnp.where(h >= 0, h, _SLOPE * h)
    o_ref[...] = ad_ref[0, :] * h + ad_ref[1, :]


def _apply_norm(h, ad, n):
    return pl.pallas_call(
        _apply_body,
        grid=(n // _BN,),
        in_specs=[
            pl.BlockSpec((_BN, _C), lambda i: (i, 0)),
            pl.BlockSpec((2, _C), lambda i: (0, 0)),
        ],
        out_specs=pl.BlockSpec((_BN, _C), lambda i: (i, 0)),
        out_shape=jax.ShapeDtypeStruct((n, _C), jnp.float32),
    )(h, ad)


def _idx_body(src_ref, o_ref, *, n):
    o_ref[...] = src_ref[...] + pl.program_id(0) * n


def _flat_gather_idx(src, n):
    K, P = src.shape
    out = pl.pallas_call(
        functools.partial(_idx_body, n=n),
        grid=(K,),
        in_specs=[pl.BlockSpec((1, 1, P), lambda k: (k, 0, 0))],
        out_specs=pl.BlockSpec((1, 1, P), lambda k: (k, 0, 0)),
        out_shape=jax.ShapeDtypeStruct((K, 1, P), jnp.int32),
    )(src.reshape(K, 1, P))
    return out.reshape(K * P)


_CH = 6400
_Q = 384
_BL = 2048
_NSUB = 16


def _sc_scatter_body(y_hbm, g_hbm, d_hbm, init_hbm, out_hbm,
                     dvec, gvec, dvec2, gvec2, cga, cla, stage, shacc,
                     sem, sem2, sem3, sem4,
                     *, n_pad, per_tile):
    core = lax.axis_index("c")
    sub = lax.axis_index("s")
    nchunks = n_pad // _CH
    rpt = _CH // _NSUB
    iota = lax.iota(jnp.int32, 16)

    def flush():
        ng = _Q // 128
        gath = [pltpu.async_copy(y_hbm.at[cga.at[j]],
                                 stage.at[pl.ds(j * 128, 128)], sem)
                for j in range(ng)]
        scat = []
        for j in range(ng):
            gath[j].wait()
            scat.append(pltpu.async_copy(stage.at[pl.ds(j * 128, 128)],
                                         shacc.at[cla.at[j]], sem2,
                                         add=True))
        for s in scat:
            s.wait()

    for q in range(nchunks // 2):
        chunk = 2 * q + core
        base = chunk * _CH
        pltpu.sync_copy(init_hbm.at[pl.ds(base + sub * rpt, rpt)],
                        shacc.at[pl.ds(sub * rpt, rpt)])
        plsc.subcore_barrier()

        def scan_buf(bd, bg, fill):
            def body(i, fill):
                d = bd[pl.ds(i * 16, 16)]
                g = bg[pl.ds(i * 16, 16)]
                local = d - base
                m = local.astype(jnp.uint32) < jnp.uint32(_CH)
                mi = m.astype(jnp.int32)
                pos = fill + plsc.cumsum(mi) - 1
                plsc.store_scatter(cga, [pos >> 7, pos & 127], g, mask=m)
                plsc.store_scatter(cla, [pos >> 7, pos & 127], local,
                                   mask=m)
                fill = fill + jnp.sum(mi)

                def do_flush():
                    flush()
                    cga[0, pl.ds(0, 16)] = cga[_Q // 128, pl.ds(0, 16)]
                    cla[0, pl.ds(0, 16)] = cla[_Q // 128, pl.ds(0, 16)]

                pl.when(fill >= _Q)(do_flush)
                return jnp.where(fill >= _Q, fill - _Q, fill)

            return lax.fori_loop(0, _BL // 16, body, fill)

        def start_blk(b, bd, bg, s):
            off = sub * per_tile + b * _BL
            pltpu.async_copy(d_hbm.at[pl.ds(off, _BL)], bd, s)
            pltpu.async_copy(g_hbm.at[pl.ds(off, _BL)], bg, s)

        def wait_blk(bd, bg, s):
            pltpu.make_async_copy(d_hbm.at[pl.ds(0, _BL)], bd, s).wait()
            pltpu.make_async_copy(g_hbm.at[pl.ds(0, _BL)], bg, s).wait()

        nb = per_tile // _BL
        start_blk(0, dvec, gvec, sem3)

        def pair(t, fill):
            b = 2 * t
            wait_blk(dvec, gvec, sem3)
            start_blk(b + 1, dvec2, gvec2, sem4)
            fill = scan_buf(dvec, gvec, fill)
            wait_blk(dvec2, gvec2, sem4)

            @pl.when(b + 2 < nb)
            def _():
                start_blk(b + 2, dvec, gvec, sem3)

            return scan_buf(dvec2, gvec2, fill)

        fill = lax.fori_loop(0, nb // 2, pair, jnp.int32(0))

        zero16 = jnp.zeros((16,), jnp.int32)
        dump16 = jnp.full((16,), _CH, jnp.int32)
        for j in range(_Q // 16):
            posj = fill + j * 16 + iota
            mj = posj < _Q + 16
            plsc.store_scatter(cga, [posj >> 7, posj & 127], zero16, mask=mj)
            plsc.store_scatter(cla, [posj >> 7, posj & 127], dump16, mask=mj)
        flush()
        plsc.subcore_barrier()
        pltpu.sync_copy(shacc.at[pl.ds(sub * rpt, rpt)],
                        out_hbm.at[pl.ds(base + sub * rpt, rpt)])
        plsc.subcore_barrier()


def _sc_gather_scatter_add(Y2d, gidx, dst, init):
    n_pad = init.shape[0]
    kp_pad = gidx.shape[0]
    per_tile = kp_pad // _NSUB
    mesh = plsc.VectorSubcoreMesh(core_axis_name="c", subcore_axis_name="s")
    cp = pltpu.CompilerParams()
    if "needs_layout_passes" in pltpu.CompilerParams.__dataclass_fields__:
        cp = dataclasses.replace(cp, needs_layout_passes=False)
    kern = pl.kernel(
        functools.partial(_sc_scatter_body, n_pad=n_pad, per_tile=per_tile),
        mesh=mesh,
        out_type=jax.ShapeDtypeStruct((n_pad, _C), jnp.float32),
        scratch_types=[
            pltpu.VMEM((_BL,), jnp.int32),
            pltpu.VMEM((_BL,), jnp.int32),
            pltpu.VMEM((_BL,), jnp.int32),
            pltpu.VMEM((_BL,), jnp.int32),
            pltpu.VMEM((_Q // 128 + 1, 128), jnp.int32),
            pltpu.VMEM((_Q // 128 + 1, 128), jnp.int32),
            pltpu.VMEM((_Q, _C), jnp.float32),
            pltpu.VMEM_SHARED((_CH + 8, _C), jnp.float32),
            pltpu.SemaphoreType.DMA,
            pltpu.SemaphoreType.DMA,
            pltpu.SemaphoreType.DMA,
            pltpu.SemaphoreType.DMA,
        ],
        compiler_params=cp,
    )
    return kern(Y2d, gidx, dst, init)


def _pad_pairs(idx, pad_to, fill):
    flat = idx.reshape(-1)
    return jnp.pad(flat, (0, pad_to - flat.shape[0]), constant_values=fill)


def _gather_scatter_add(Y, src, dst, init):
    K, n, Cc = Y.shape
    kp = K * src.shape[1]
    kp_pad = -(-kp // (_NSUB * 2 * _BL)) * (_NSUB * 2 * _BL)
    gidx = _pad_pairs(_flat_gather_idx(src, n), kp_pad, 0)
    dstf = _pad_pairs(dst, kp_pad, -(2 ** 30))
    return _sc_gather_scatter_add(Y.reshape(K * n, Cc), gidx, dstf, init)


_ID_AD = None


def _pad_rows(a, n_pad):
    return jnp.pad(a, ((0, n_pad - a.shape[0]), (0, 0)))


def kernel(x, skip, W1, Wup, W2, W3, W4, g1, b1, g2, b2, g3, b3, g4, b4,
           conv1_src, conv1_dst, up_src, up_dst, conv2_src, conv2_dst,
           conv3_src, conv3_dst, conv4_src, conv4_dst):
    n_in_pad = -(-_N_IN // (2 * _CH)) * (2 * _CH)
    n_out_pad = -(-_N_OUT // (2 * _CH)) * (2 * _CH)
    zero_in = jnp.zeros((n_in_pad, _C), jnp.float32)
    zero_out = jnp.zeros((n_out_pad, _C), jnp.float32)
    id_ad = jnp.zeros((2, _C), jnp.float32)

    Y1 = _dense_gemm(x, W1, id_ad, normed=False, n=_N_IN)
    h1 = _gather_scatter_add(Y1, conv1_src, conv1_dst, zero_in)
    ad1 = _lrelu_bn_stats(h1, g1, b1, n=_N_IN)

    Yup = _dense_gemm(h1, Wup, ad1, normed=True, n=_N_IN)
    h2 = _gather_scatter_add(Yup, up_src, up_dst, _pad_rows(skip, n_out_pad))

    Y2 = _dense_gemm(h2, W2, id_ad, normed=False, n=_N_OUT)
    h3 = _gather_scatter_add(Y2, conv2_src, conv2_dst, zero_out)
    ad2 = _lrelu_bn_stats(h3, g2, b2, n=_N_OUT)

    Y3 = _dense_gemm(h3, W3, ad2, normed=True, n=_N_OUT)
    h4 = _gather_scatter_add(Y3, conv3_src, conv3_dst, zero_out)
    ad3 = _lrelu_bn_stats(h4, g3, b3, n=_N_OUT)

    Y4 = _dense_gemm(h4, W4, ad3, normed=True, n=_N_OUT)
    h5 = _gather_scatter_add(Y4, conv4_src, conv4_dst, zero_out)
    ad4 = _lrelu_bn_stats(h5, g4, b4, n=_N_OUT)

    return _apply_norm(h5, ad4, n=_N_OUT)


# speedup vs baseline: 2.0057x; 2.0057x over previous
import dataclasses
import functools

import jax
import jax.numpy as jnp
from jax import lax
from jax.experimental import pallas as pl
from jax.experimental.pallas import tpu as pltpu
from jax.experimental.pallas import tpu_sc as plsc

_N_IN = 50000
_N_OUT = 100000
_C = 128
_EPS = 1e-5
_SLOPE = 0.01
_BN = 10000


def _stats_body(h_ref, g_ref, b_ref, o_ref, acc_ref, *, nrows, nblocks):
    i = pl.program_id(0)

    @pl.when(i == 0)
    def _():
        acc_ref[...] = jnp.zeros_like(acc_ref)

    h = h_ref[...]
    h = jnp.where(h >= 0, h, _SLOPE * h)
    acc_ref[0, :] += jnp.sum(h, axis=0)
    acc_ref[1, :] += jnp.sum(h * h, axis=0)

    @pl.when(i == nblocks - 1)
    def _():
        m = acc_ref[0, :] / nrows
        v = acc_ref[1, :] / nrows - m * m
        a = g_ref[0, :] * jax.lax.rsqrt(v + _EPS)
        o_ref[0, :] = a
        o_ref[1, :] = b_ref[0, :] - m * a


def _lrelu_bn_stats(h, g, b, n):
    nblocks = n // _BN
    return pl.pallas_call(
        functools.partial(_stats_body, nrows=float(n), nblocks=nblocks),
        grid=(nblocks,),
        in_specs=[
            pl.BlockSpec((_BN, _C), lambda i: (i, 0)),
            pl.BlockSpec((1, _C), lambda i: (0, 0)),
            pl.BlockSpec((1, _C), lambda i: (0, 0)),
        ],
        out_specs=pl.BlockSpec((2, _C), lambda i: (0, 0)),
        out_shape=jax.ShapeDtypeStruct((2, _C), jnp.float32),
        scratch_shapes=[pltpu.VMEM((2, _C), jnp.float32)],
    )(h, g.reshape(1, _C), b.reshape(1, _C))


def _gemm_body(feat_ref, w_ref, ad_ref, y_ref, *, normed):
    k = pl.program_id(1)
    f = feat_ref[...]
    wk = w_ref[k]
    hi = jax.lax.Precision.DEFAULT
    if normed:
        a = ad_ref[0, :]
        d = ad_ref[1, :]
        f = jnp.where(f >= 0, f, _SLOPE * f)
        wk = a[:, None] * wk
        t = jnp.dot(d[None, :], w_ref[k], precision=hi,
                    preferred_element_type=jnp.float32)
        y_ref[0] = jnp.dot(f, wk, precision=hi,
                           preferred_element_type=jnp.float32) + t
    else:
        y_ref[0] = jnp.dot(f, wk, precision=hi,
                           preferred_element_type=jnp.float32)


def _dense_gemm(feat, W, ad, normed, n):
    K = W.shape[0]
    nblocks = n // _BN
    return pl.pallas_call(
        functools.partial(_gemm_body, normed=normed),
        grid=(nblocks, K),
        in_specs=[
            pl.BlockSpec((_BN, _C), lambda i, k: (i, 0)),
            pl.BlockSpec((K, _C, _C), lambda i, k: (0, 0, 0)),
            pl.BlockSpec((2, _C), lambda i, k: (0, 0)),
        ],
        out_specs=pl.BlockSpec((1, _BN, _C), lambda i, k: (k, i, 0)),
        out_shape=jax.ShapeDtypeStruct((K, n, _C), jnp.float32),
    )(feat, W, ad)


def _apply_body(h_ref, ad_ref, o_ref):
    h = h_ref[...]
    h = jnp.where(h >= 0, h, _SLOPE * h)
    o_ref[...] = ad_ref[0, :] * h + ad_ref[1, :]


def _apply_norm(h, ad, n):
    return pl.pallas_call(
        _apply_body,
        grid=(n // _BN,),
        in_specs=[
            pl.BlockSpec((_BN, _C), lambda i: (i, 0)),
            pl.BlockSpec((2, _C), lambda i: (0, 0)),
        ],
        out_specs=pl.BlockSpec((_BN, _C), lambda i: (i, 0)),
        out_shape=jax.ShapeDtypeStruct((n, _C), jnp.float32),
    )(h, ad)


def _idx_body(src_ref, o_ref, *, n):
    o_ref[...] = src_ref[...] + pl.program_id(0) * n


def _flat_gather_idx(src, n):
    K, P = src.shape
    out = pl.pallas_call(
        functools.partial(_idx_body, n=n),
        grid=(K,),
        in_specs=[pl.BlockSpec((1, 1, P), lambda k: (k, 0, 0))],
        out_specs=pl.BlockSpec((1, 1, P), lambda k: (k, 0, 0)),
        out_shape=jax.ShapeDtypeStruct((K, 1, P), jnp.int32),
    )(src.reshape(K, 1, P))
    return out.reshape(K * P)


_CH = 11264
_Q = 256
_BL = 2048
_NSUB = 16


def _sc_scatter_body(y_hbm, g_hbm, d_hbm, init_hbm, out_hbm,
                     dvec, gvec, cga, cla, stage, shacc, sem, sem2,
                     *, n_pad, per_tile):
    core = lax.axis_index("c")
    sub = lax.axis_index("s")
    nchunks = n_pad // _CH
    rpt = _CH // _NSUB
    iota = lax.iota(jnp.int32, 16)

    def flush():
        ng = _Q // 128
        gath = [pltpu.async_copy(y_hbm.at[cga.at[j]],
                                 stage.at[pl.ds(j * 128, 128)], sem)
                for j in range(ng)]
        scat = []
        for j in range(ng):
            gath[j].wait()
            scat.append(pltpu.async_copy(stage.at[pl.ds(j * 128, 128)],
                                         shacc.at[cla.at[j]], sem2,
                                         add=True))
        for s in scat:
            s.wait()

    for q in range(nchunks // 2):
        chunk = 2 * q + core
        base = chunk * _CH
        pltpu.sync_copy(init_hbm.at[pl.ds(base + sub * rpt, rpt)],
                        shacc.at[pl.ds(sub * rpt, rpt)])
        plsc.subcore_barrier()

        def scan_buf(bd, bg, fill):
            def body(i, fill):
                d = bd[pl.ds(i * 16, 16)]
                g = bg[pl.ds(i * 16, 16)]
                local = d - base
                m = local.astype(jnp.uint32) < jnp.uint32(_CH)
                mi = m.astype(jnp.int32)
                pos = fill + plsc.cumsum(mi) - 1
                plsc.store_scatter(cga, [pos >> 7, pos & 127], g, mask=m)
                plsc.store_scatter(cla, [pos >> 7, pos & 127], local,
                                   mask=m)
                fill = fill + jnp.sum(mi)

                def do_flush():
                    flush()
                    cga[0, pl.ds(0, 16)] = cga[_Q // 128, pl.ds(0, 16)]
                    cla[0, pl.ds(0, 16)] = cla[_Q // 128, pl.ds(0, 16)]

                pl.when(fill >= _Q)(do_flush)
                return jnp.where(fill >= _Q, fill - _Q, fill)

            return lax.fori_loop(0, _BL // 16, body, fill)

        def blk(b, fill):
            off = sub * per_tile + b * _BL
            pltpu.sync_copy(d_hbm.at[pl.ds(off, _BL)], dvec)
            pltpu.sync_copy(g_hbm.at[pl.ds(off, _BL)], gvec)
            return scan_buf(dvec, gvec, fill)

        fill = lax.fori_loop(0, per_tile // _BL, blk, jnp.int32(0))

        zero16 = jnp.zeros((16,), jnp.int32)
        dump16 = jnp.full((16,), _CH, jnp.int32)
        for j in range(_Q // 16):
            posj = fill + j * 16 + iota
            mj = posj < _Q + 16
            plsc.store_scatter(cga, [posj >> 7, posj & 127], zero16, mask=mj)
            plsc.store_scatter(cla, [posj >> 7, posj & 127], dump16, mask=mj)
        flush()
        plsc.subcore_barrier()
        pltpu.sync_copy(shacc.at[pl.ds(sub * rpt, rpt)],
                        out_hbm.at[pl.ds(base + sub * rpt, rpt)])
        plsc.subcore_barrier()


def _sc_gather_scatter_add(Y2d, gidx, dst, init):
    n_pad = init.shape[0]
    kp_pad = gidx.shape[0]
    per_tile = kp_pad // _NSUB
    mesh = plsc.VectorSubcoreMesh(core_axis_name="c", subcore_axis_name="s")
    cp = pltpu.CompilerParams()
    if "needs_layout_passes" in pltpu.CompilerParams.__dataclass_fields__:
        cp = dataclasses.replace(cp, needs_layout_passes=False)
    kern = pl.kernel(
        functools.partial(_sc_scatter_body, n_pad=n_pad, per_tile=per_tile),
        mesh=mesh,
        out_type=jax.ShapeDtypeStruct((n_pad, _C), jnp.float32),
        scratch_types=[
            pltpu.VMEM((_BL,), jnp.int32),
            pltpu.VMEM((_BL,), jnp.int32),
            pltpu.VMEM((_Q // 128 + 1, 128), jnp.int32),
            pltpu.VMEM((_Q // 128 + 1, 128), jnp.int32),
            pltpu.VMEM((_Q, _C), jnp.float32),
            pltpu.VMEM_SHARED((_CH + 8, _C), jnp.float32),
            pltpu.SemaphoreType.DMA,
            pltpu.SemaphoreType.DMA,
        ],
        compiler_params=cp,
    )
    return kern(Y2d, gidx, dst, init)


def _pad_pairs(idx, pad_to, fill):
    flat = idx.reshape(-1)
    return jnp.pad(flat, (0, pad_to - flat.shape[0]), constant_values=fill)


def _gather_scatter_add(Y, src, dst, init):
    K, n, Cc = Y.shape
    kp = K * src.shape[1]
    kp_pad = -(-kp // (_NSUB * _BL)) * (_NSUB * _BL)
    gidx = _pad_pairs(_flat_gather_idx(src, n), kp_pad, 0)
    dstf = _pad_pairs(dst, kp_pad, -(2 ** 30))
    return _sc_gather_scatter_add(Y.reshape(K * n, Cc), gidx, dstf, init)


_ID_AD = None


def _pad_rows(a, n_pad):
    return jnp.pad(a, ((0, n_pad - a.shape[0]), (0, 0)))


def kernel(x, skip, W1, Wup, W2, W3, W4, g1, b1, g2, b2, g3, b3, g4, b4,
           conv1_src, conv1_dst, up_src, up_dst, conv2_src, conv2_dst,
           conv3_src, conv3_dst, conv4_src, conv4_dst):
    n_in_pad = -(-_N_IN // (2 * _CH)) * (2 * _CH)
    n_out_pad = -(-_N_OUT // (2 * _CH)) * (2 * _CH)
    zero_in = jnp.zeros((n_in_pad, _C), jnp.float32)
    zero_out = jnp.zeros((n_out_pad, _C), jnp.float32)
    id_ad = jnp.zeros((2, _C), jnp.float32)

    Y1 = _dense_gemm(x, W1, id_ad, normed=False, n=_N_IN)
    h1 = _gather_scatter_add(Y1, conv1_src, conv1_dst, zero_in)
    ad1 = _lrelu_bn_stats(h1, g1, b1, n=_N_IN)

    Yup = _dense_gemm(h1, Wup, ad1, normed=True, n=_N_IN)
    h2 = _gather_scatter_add(Yup, up_src, up_dst, _pad_rows(skip, n_out_pad))

    Y2 = _dense_gemm(h2, W2, id_ad, normed=False, n=_N_OUT)
    h3 = _gather_scatter_add(Y2, conv2_src, conv2_dst, zero_out)
    ad2 = _lrelu_bn_stats(h3, g2, b2, n=_N_OUT)

    Y3 = _dense_gemm(h3, W3, ad2, normed=True, n=_N_OUT)
    h4 = _gather_scatter_add(Y3, conv3_src, conv3_dst, zero_out)
    ad3 = _lrelu_bn_stats(h4, g3, b3, n=_N_OUT)

    Y4 = _dense_gemm(h4, W4, ad3, normed=True, n=_N_OUT)
    h5 = _gather_scatter_add(Y4, conv4_src, conv4_dst, zero_out)
    ad4 = _lrelu_bn_stats(h5, g4, b4, n=_N_OUT)

    return _apply_norm(h5, ad4, n=_N_OUT)


# speedup vs baseline: 2.0450x; 1.0196x over previous
import dataclasses
import functools

import jax
import jax.numpy as jnp
from jax import lax
from jax.experimental import pallas as pl
from jax.experimental.pallas import tpu as pltpu
from jax.experimental.pallas import tpu_sc as plsc

_N_IN = 50000
_N_OUT = 100000
_C = 128
_EPS = 1e-5
_SLOPE = 0.01
_BN = 10000


def _stats_body(h_ref, g_ref, b_ref, o_ref, acc_ref, *, nrows, nblocks):
    i = pl.program_id(0)

    @pl.when(i == 0)
    def _():
        acc_ref[...] = jnp.zeros_like(acc_ref)

    h = h_ref[...]
    h = jnp.where(h >= 0, h, _SLOPE * h)
    acc_ref[0, :] += jnp.sum(h, axis=0)
    acc_ref[1, :] += jnp.sum(h * h, axis=0)

    @pl.when(i == nblocks - 1)
    def _():
        m = acc_ref[0, :] / nrows
        v = acc_ref[1, :] / nrows - m * m
        a = g_ref[0, :] * jax.lax.rsqrt(v + _EPS)
        o_ref[0, :] = a
        o_ref[1, :] = b_ref[0, :] - m * a


def _lrelu_bn_stats(h, g, b, n):
    nblocks = n // _BN
    return pl.pallas_call(
        functools.partial(_stats_body, nrows=float(n), nblocks=nblocks),
        grid=(nblocks,),
        in_specs=[
            pl.BlockSpec((_BN, _C), lambda i: (i, 0)),
            pl.BlockSpec((1, _C), lambda i: (0, 0)),
            pl.BlockSpec((1, _C), lambda i: (0, 0)),
        ],
        out_specs=pl.BlockSpec((2, _C), lambda i: (0, 0)),
        out_shape=jax.ShapeDtypeStruct((2, _C), jnp.float32),
        scratch_shapes=[pltpu.VMEM((2, _C), jnp.float32)],
    )(h, g.reshape(1, _C), b.reshape(1, _C))


def _gemm_body(feat_ref, w_ref, ad_ref, y_ref, *, normed):
    k = pl.program_id(1)
    f = feat_ref[...]
    wk = w_ref[k]
    hi = jax.lax.Precision.DEFAULT
    if normed:
        a = ad_ref[0, :]
        d = ad_ref[1, :]
        f = jnp.where(f >= 0, f, _SLOPE * f)
        wk = a[:, None] * wk
        t = jnp.dot(d[None, :], w_ref[k], precision=hi,
                    preferred_element_type=jnp.float32)
        y_ref[0] = jnp.dot(f, wk, precision=hi,
                           preferred_element_type=jnp.float32) + t
    else:
        y_ref[0] = jnp.dot(f, wk, precision=hi,
                           preferred_element_type=jnp.float32)


def _dense_gemm(feat, W, ad, normed, n):
    K = W.shape[0]
    nblocks = n // _BN
    return pl.pallas_call(
        functools.partial(_gemm_body, normed=normed),
        grid=(nblocks, K),
        in_specs=[
            pl.BlockSpec((_BN, _C), lambda i, k: (i, 0)),
            pl.BlockSpec((K, _C, _C), lambda i, k: (0, 0, 0)),
            pl.BlockSpec((2, _C), lambda i, k: (0, 0)),
        ],
        out_specs=pl.BlockSpec((1, _BN, _C), lambda i, k: (k, i, 0)),
        out_shape=jax.ShapeDtypeStruct((K, n, _C), jnp.float32),
    )(feat, W, ad)


def _apply_body(h_ref, ad_ref, o_ref):
    h = h_ref[...]
    h = jnp.where(h >= 0, h, _SLOPE * h)
    o_ref[...] = ad_ref[0, :] * h + ad_ref[1, :]


def _apply_norm(h, ad, n):
    return pl.pallas_call(
        _apply_body,
        grid=(n // _BN,),
        in_specs=[
            pl.BlockSpec((_BN, _C), lambda i: (i, 0)),
            pl.BlockSpec((2, _C), lambda i: (0, 0)),
        ],
        out_specs=pl.BlockSpec((_BN, _C), lambda i: (i, 0)),
        out_shape=jax.ShapeDtypeStruct((n, _C), jnp.float32),
    )(h, ad)


def _idx_body(src_ref, o_ref, *, n):
    o_ref[...] = src_ref[...] + pl.program_id(0) * n


def _flat_gather_idx(src, n):
    K, P = src.shape
    out = pl.pallas_call(
        functools.partial(_idx_body, n=n),
        grid=(K,),
        in_specs=[pl.BlockSpec((1, 1, P), lambda k: (k, 0, 0))],
        out_specs=pl.BlockSpec((1, 1, P), lambda k: (k, 0, 0)),
        out_shape=jax.ShapeDtypeStruct((K, 1, P), jnp.int32),
    )(src.reshape(K, 1, P))
    return out.reshape(K * P)


_CH = 11264
_Q = 256
_BL = 2048
_NSUB = 16


def _sc_scatter_body(y_hbm, g_hbm, d_hbm, init_hbm, out_hbm,
                     dvec, gvec, cga, cla, stage, shacc, sem, sem2,
                     *, n_pad, per_tile):
    core = lax.axis_index("c")
    sub = lax.axis_index("s")
    nchunks = n_pad // _CH
    rpt = _CH // _NSUB
    iota = lax.iota(jnp.int32, 16)

    def flush():
        ng = _Q // 128
        gath = [pltpu.async_copy(y_hbm.at[cga.at[j]],
                                 stage.at[pl.ds(j * 128, 128)], sem)
                for j in range(ng)]
        scat = []
        for j in range(ng):
            gath[j].wait()
            scat.append(pltpu.async_copy(stage.at[pl.ds(j * 128, 128)],
                                         shacc.at[cla.at[j]], sem2,
                                         add=True))
        for s in scat:
            s.wait()

    for q in range(nchunks // 2):
        chunk = 2 * q + core
        base = chunk * _CH
        pltpu.sync_copy(init_hbm.at[pl.ds(base + sub * rpt, rpt)],
                        shacc.at[pl.ds(sub * rpt, rpt)])
        plsc.subcore_barrier()

        def scan_buf(bd, bg, fill):
            def body(i, fill):
                d = bd[pl.ds(i * 16, 16)]
                g = bg[pl.ds(i * 16, 16)]
                local = d - base
                m = local.astype(jnp.uint32) < jnp.uint32(_CH)
                mi = m.astype(jnp.int32)
                pos = fill + plsc.cumsum(mi) - 1
                plsc.store_scatter(cga, [pos >> 7, pos & 127], g, mask=m)
                plsc.store_scatter(cla, [pos >> 7, pos & 127], local,
                                   mask=m)
                fill = fill + jnp.sum(mi)

                def do_flush():
                    flush()
                    cga[0, pl.ds(0, 16)] = cga[_Q // 128, pl.ds(0, 16)]
                    cla[0, pl.ds(0, 16)] = cla[_Q // 128, pl.ds(0, 16)]

                pl.when(fill >= _Q)(do_flush)
                return jnp.where(fill >= _Q, fill - _Q, fill)

            return lax.fori_loop(0, _BL // 16, body, fill)

        def blk(b, fill):
            off = sub * per_tile + b * _BL
            h1 = pltpu.async_copy(d_hbm.at[pl.ds(off, _BL)], dvec, sem2)
            h2 = pltpu.async_copy(g_hbm.at[pl.ds(off, _BL)], gvec, sem2)
            h1.wait()
            h2.wait()
            return scan_buf(dvec, gvec, fill)

        fill = lax.fori_loop(0, per_tile // _BL, blk, jnp.int32(0))

        zero16 = jnp.zeros((16,), jnp.int32)
        dump16 = jnp.full((16,), _CH, jnp.int32)
        for j in range(_Q // 16):
            posj = fill + j * 16 + iota
            mj = posj < _Q + 16
            plsc.store_scatter(cga, [posj >> 7, posj & 127], zero16, mask=mj)
            plsc.store_scatter(cla, [posj >> 7, posj & 127], dump16, mask=mj)
        flush()
        plsc.subcore_barrier()
        pltpu.sync_copy(shacc.at[pl.ds(sub * rpt, rpt)],
                        out_hbm.at[pl.ds(base + sub * rpt, rpt)])
        plsc.subcore_barrier()


def _sc_gather_scatter_add(Y2d, gidx, dst, init):
    n_pad = init.shape[0]
    kp_pad = gidx.shape[0]
    per_tile = kp_pad // _NSUB
    mesh = plsc.VectorSubcoreMesh(core_axis_name="c", subcore_axis_name="s")
    cp = pltpu.CompilerParams()
    if "needs_layout_passes" in pltpu.CompilerParams.__dataclass_fields__:
        cp = dataclasses.replace(cp, needs_layout_passes=False)
    kern = pl.kernel(
        functools.partial(_sc_scatter_body, n_pad=n_pad, per_tile=per_tile),
        mesh=mesh,
        out_type=jax.ShapeDtypeStruct((n_pad, _C), jnp.float32),
        scratch_types=[
            pltpu.VMEM((_BL,), jnp.int32),
            pltpu.VMEM((_BL,), jnp.int32),
            pltpu.VMEM((_Q // 128 + 1, 128), jnp.int32),
            pltpu.VMEM((_Q // 128 + 1, 128), jnp.int32),
            pltpu.VMEM((_Q, _C), jnp.float32),
            pltpu.VMEM_SHARED((_CH + 8, _C), jnp.float32),
            pltpu.SemaphoreType.DMA,
            pltpu.SemaphoreType.DMA,
        ],
        compiler_params=cp,
    )
    return kern(Y2d, gidx, dst, init)


def _pad_pairs(idx, pad_to, fill):
    flat = idx.reshape(-1)
    return jnp.pad(flat, (0, pad_to - flat.shape[0]), constant_values=fill)


def _gather_scatter_add(Y, src, dst, init):
    K, n, Cc = Y.shape
    kp = K * src.shape[1]
    kp_pad = -(-kp // (_NSUB * _BL)) * (_NSUB * _BL)
    gidx = _pad_pairs(_flat_gather_idx(src, n), kp_pad, 0)
    dstf = _pad_pairs(dst, kp_pad, -(2 ** 30))
    return _sc_gather_scatter_add(Y.reshape(K * n, Cc), gidx, dstf, init)


_ID_AD = None


def _pad_rows(a, n_pad):
    return jnp.pad(a, ((0, n_pad - a.shape[0]), (0, 0)))


def kernel(x, skip, W1, Wup, W2, W3, W4, g1, b1, g2, b2, g3, b3, g4, b4,
           conv1_src, conv1_dst, up_src, up_dst, conv2_src, conv2_dst,
           conv3_src, conv3_dst, conv4_src, conv4_dst):
    n_in_pad = -(-_N_IN // (2 * _CH)) * (2 * _CH)
    n_out_pad = -(-_N_OUT // (2 * _CH)) * (2 * _CH)
    zero_in = jnp.zeros((n_in_pad, _C), jnp.float32)
    zero_out = jnp.zeros((n_out_pad, _C), jnp.float32)
    id_ad = jnp.zeros((2, _C), jnp.float32)

    Y1 = _dense_gemm(x, W1, id_ad, normed=False, n=_N_IN)
    h1 = _gather_scatter_add(Y1, conv1_src, conv1_dst, zero_in)
    ad1 = _lrelu_bn_stats(h1, g1, b1, n=_N_IN)

    Yup = _dense_gemm(h1, Wup, ad1, normed=True, n=_N_IN)
    h2 = _gather_scatter_add(Yup, up_src, up_dst, _pad_rows(skip, n_out_pad))

    Y2 = _dense_gemm(h2, W2, id_ad, normed=False, n=_N_OUT)
    h3 = _gather_scatter_add(Y2, conv2_src, conv2_dst, zero_out)
    ad2 = _lrelu_bn_stats(h3, g2, b2, n=_N_OUT)

    Y3 = _dense_gemm(h3, W3, ad2, normed=True, n=_N_OUT)
    h4 = _gather_scatter_add(Y3, conv3_src, conv3_dst, zero_out)
    ad3 = _lrelu_bn_stats(h4, g3, b3, n=_N_OUT)

    Y4 = _dense_gemm(h4, W4, ad3, normed=True, n=_N_OUT)
    h5 = _gather_scatter_add(Y4, conv4_src, conv4_dst, zero_out)
    ad4 = _lrelu_bn_stats(h5, g4, b4, n=_N_OUT)

    return _apply_norm(h5, ad4, n=_N_OUT)


# speedup vs baseline: 2.2003x; 1.0760x over previous
import dataclasses
import functools

import jax
import jax.numpy as jnp
from jax import lax
from jax.experimental import pallas as pl
from jax.experimental.pallas import tpu as pltpu
from jax.experimental.pallas import tpu_sc as plsc

_N_IN = 50000
_N_OUT = 100000
_C = 128
_EPS = 1e-5
_SLOPE = 0.01
_BN = 10000


def _stats_body(h_ref, g_ref, b_ref, o_ref, acc_ref, *, nrows, nblocks):
    i = pl.program_id(0)

    @pl.when(i == 0)
    def _():
        acc_ref[...] = jnp.zeros_like(acc_ref)

    h = h_ref[...]
    h = jnp.where(h >= 0, h, _SLOPE * h)
    acc_ref[0, :] += jnp.sum(h, axis=0)
    acc_ref[1, :] += jnp.sum(h * h, axis=0)

    @pl.when(i == nblocks - 1)
    def _():
        m = acc_ref[0, :] / nrows
        v = acc_ref[1, :] / nrows - m * m
        a = g_ref[0, :] * jax.lax.rsqrt(v + _EPS)
        o_ref[0, :] = a
        o_ref[1, :] = b_ref[0, :] - m * a


def _lrelu_bn_stats(h, g, b, n):
    nblocks = n // _BN
    return pl.pallas_call(
        functools.partial(_stats_body, nrows=float(n), nblocks=nblocks),
        grid=(nblocks,),
        in_specs=[
            pl.BlockSpec((_BN, _C), lambda i: (i, 0)),
            pl.BlockSpec((1, _C), lambda i: (0, 0)),
            pl.BlockSpec((1, _C), lambda i: (0, 0)),
        ],
        out_specs=pl.BlockSpec((2, _C), lambda i: (0, 0)),
        out_shape=jax.ShapeDtypeStruct((2, _C), jnp.float32),
        scratch_shapes=[pltpu.VMEM((2, _C), jnp.float32)],
    )(h, g.reshape(1, _C), b.reshape(1, _C))


def _gemm_body(feat_ref, w_ref, ad_ref, y_ref, *, normed):
    k = pl.program_id(1)
    f = feat_ref[...]
    wk = w_ref[k]
    hi = jax.lax.Precision.DEFAULT
    if normed:
        a = ad_ref[0, :]
        d = ad_ref[1, :]
        f = jnp.where(f >= 0, f, _SLOPE * f)
        wk = a[:, None] * wk
        t = jnp.dot(d[None, :], w_ref[k], precision=hi,
                    preferred_element_type=jnp.float32)
        y_ref[0] = jnp.dot(f, wk, precision=hi,
                           preferred_element_type=jnp.float32) + t
    else:
        y_ref[0] = jnp.dot(f, wk, precision=hi,
                           preferred_element_type=jnp.float32)


def _dense_gemm(feat, W, ad, normed, n):
    K = W.shape[0]
    nblocks = n // _BN
    return pl.pallas_call(
        functools.partial(_gemm_body, normed=normed),
        grid=(nblocks, K),
        in_specs=[
            pl.BlockSpec((_BN, _C), lambda i, k: (i, 0)),
            pl.BlockSpec((K, _C, _C), lambda i, k: (0, 0, 0)),
            pl.BlockSpec((2, _C), lambda i, k: (0, 0)),
        ],
        out_specs=pl.BlockSpec((1, _BN, _C), lambda i, k: (k, i, 0)),
        out_shape=jax.ShapeDtypeStruct((K, n, _C), jnp.float32),
    )(feat, W, ad)


def _apply_body(h_ref, ad_ref, o_ref):
    h = h_ref[...]
    h = jnp.where(h >= 0, h, _SLOPE * h)
    o_ref[...] = ad_ref[0, :] * h + ad_ref[1, :]


def _apply_norm(h, ad, n):
    return pl.pallas_call(
        _apply_body,
        grid=(n // _BN,),
        in_specs=[
            pl.BlockSpec((_BN, _C), lambda i: (i, 0)),
            pl.BlockSpec((2, _C), lambda i: (0, 0)),
        ],
        out_specs=pl.BlockSpec((_BN, _C), lambda i: (i, 0)),
        out_shape=jax.ShapeDtypeStruct((n, _C), jnp.float32),
    )(h, ad)


def _idx_body(src_ref, o_ref, *, n):
    o_ref[...] = src_ref[...] + pl.program_id(0) * n


def _flat_gather_idx(src, n):
    K, P = src.shape
    out = pl.pallas_call(
        functools.partial(_idx_body, n=n),
        grid=(K,),
        in_specs=[pl.BlockSpec((1, 1, P), lambda k: (k, 0, 0))],
        out_specs=pl.BlockSpec((1, 1, P), lambda k: (k, 0, 0)),
        out_shape=jax.ShapeDtypeStruct((K, 1, P), jnp.int32),
    )(src.reshape(K, 1, P))
    return out.reshape(K * P)


_CH = 11264
_Q = 256
_BL = 2048
_NSUB = 16


def _sc_scatter_body(y_hbm, g_hbm, d_hbm, init_hbm, out_hbm,
                     dvec, gvec, cga, cla, stage, shacc, sem, sem2,
                     *, n_pad, per_tile):
    core = lax.axis_index("c")
    sub = lax.axis_index("s")
    nchunks = n_pad // _CH
    rpt = _CH // _NSUB
    iota = lax.iota(jnp.int32, 16)

    def flush():
        ng = _Q // 128
        gath = [pltpu.async_copy(y_hbm.at[cga.at[j]],
                                 stage.at[pl.ds(j * 128, 128)], sem)
                for j in range(ng)]
        scat = []
        for j in range(ng):
            gath[j].wait()
            scat.append(pltpu.async_copy(stage.at[pl.ds(j * 128, 128)],
                                         shacc.at[cla.at[j]], sem2,
                                         add=True))
        for s in scat:
            s.wait()

    for q in range(nchunks // 2):
        chunk = 2 * q + core
        base = chunk * _CH
        pltpu.sync_copy(init_hbm.at[pl.ds(base + sub * rpt, rpt)],
                        shacc.at[pl.ds(sub * rpt, rpt)])
        plsc.subcore_barrier()

        def scan_buf(bd, bg, fill):
            def quad(j, fill):
                for u in range(4):
                    i = j * 4 + u
                    d = bd[pl.ds(i * 16, 16)]
                    g = bg[pl.ds(i * 16, 16)]
                    local = d - base
                    m = local.astype(jnp.uint32) < jnp.uint32(_CH)
                    c = plsc.cumsum(m.astype(jnp.int32))
                    pos = fill + c - 1
                    plsc.store_scatter(cga, [pos >> 7, pos & 127], g,
                                       mask=m)
                    plsc.store_scatter(cla, [pos >> 7, pos & 127], local,
                                       mask=m)
                    fill = fill + c[15]

                def do_flush():
                    flush()
                    for t in range(4):
                        s = pl.ds(t * 16, 16)
                        cga[0, s] = cga[_Q // 128, s]
                        cla[0, s] = cla[_Q // 128, s]

                pl.when(fill >= _Q)(do_flush)
                return jnp.where(fill >= _Q, fill - _Q, fill)

            return lax.fori_loop(0, _BL // 64, quad, fill)

        def blk(b, fill):
            off = sub * per_tile + b * _BL
            h1 = pltpu.async_copy(d_hbm.at[pl.ds(off, _BL)], dvec, sem2)
            h2 = pltpu.async_copy(g_hbm.at[pl.ds(off, _BL)], gvec, sem2)
            h1.wait()
            h2.wait()
            return scan_buf(dvec, gvec, fill)

        fill = lax.fori_loop(0, per_tile // _BL, blk, jnp.int32(0))

        zero16 = jnp.zeros((16,), jnp.int32)
        dump16 = jnp.full((16,), _CH, jnp.int32)
        for j in range(_Q // 16):
            posj = fill + j * 16 + iota
            mj = posj < _Q + 16
            plsc.store_scatter(cga, [posj >> 7, posj & 127], zero16, mask=mj)
            plsc.store_scatter(cla, [posj >> 7, posj & 127], dump16, mask=mj)
        flush()
        plsc.subcore_barrier()
        pltpu.sync_copy(shacc.at[pl.ds(sub * rpt, rpt)],
                        out_hbm.at[pl.ds(base + sub * rpt, rpt)])
        plsc.subcore_barrier()


def _sc_gather_scatter_add(Y2d, gidx, dst, init):
    n_pad = init.shape[0]
    kp_pad = gidx.shape[0]
    per_tile = kp_pad // _NSUB
    mesh = plsc.VectorSubcoreMesh(core_axis_name="c", subcore_axis_name="s")
    cp = pltpu.CompilerParams()
    if "needs_layout_passes" in pltpu.CompilerParams.__dataclass_fields__:
        cp = dataclasses.replace(cp, needs_layout_passes=False)
    kern = pl.kernel(
        functools.partial(_sc_scatter_body, n_pad=n_pad, per_tile=per_tile),
        mesh=mesh,
        out_type=jax.ShapeDtypeStruct((n_pad, _C), jnp.float32),
        scratch_types=[
            pltpu.VMEM((_BL,), jnp.int32),
            pltpu.VMEM((_BL,), jnp.int32),
            pltpu.VMEM((_Q // 128 + 1, 128), jnp.int32),
            pltpu.VMEM((_Q // 128 + 1, 128), jnp.int32),
            pltpu.VMEM((_Q, _C), jnp.float32),
            pltpu.VMEM_SHARED((_CH + 8, _C), jnp.float32),
            pltpu.SemaphoreType.DMA,
            pltpu.SemaphoreType.DMA,
        ],
        compiler_params=cp,
    )
    return kern(Y2d, gidx, dst, init)


def _pad_pairs(idx, pad_to, fill):
    flat = idx.reshape(-1)
    return jnp.pad(flat, (0, pad_to - flat.shape[0]), constant_values=fill)


def _gather_scatter_add(Y, src, dst, init):
    K, n, Cc = Y.shape
    kp = K * src.shape[1]
    kp_pad = -(-kp // (_NSUB * _BL)) * (_NSUB * _BL)
    gidx = _pad_pairs(_flat_gather_idx(src, n), kp_pad, 0)
    dstf = _pad_pairs(dst, kp_pad, -(2 ** 30))
    return _sc_gather_scatter_add(Y.reshape(K * n, Cc), gidx, dstf, init)


_ID_AD = None


def _pad_rows(a, n_pad):
    return jnp.pad(a, ((0, n_pad - a.shape[0]), (0, 0)))


def kernel(x, skip, W1, Wup, W2, W3, W4, g1, b1, g2, b2, g3, b3, g4, b4,
           conv1_src, conv1_dst, up_src, up_dst, conv2_src, conv2_dst,
           conv3_src, conv3_dst, conv4_src, conv4_dst):
    n_in_pad = -(-_N_IN // (2 * _CH)) * (2 * _CH)
    n_out_pad = -(-_N_OUT // (2 * _CH)) * (2 * _CH)
    zero_in = jnp.zeros((n_in_pad, _C), jnp.float32)
    zero_out = jnp.zeros((n_out_pad, _C), jnp.float32)
    id_ad = jnp.zeros((2, _C), jnp.float32)

    Y1 = _dense_gemm(x, W1, id_ad, normed=False, n=_N_IN)
    h1 = _gather_scatter_add(Y1, conv1_src, conv1_dst, zero_in)
    ad1 = _lrelu_bn_stats(h1, g1, b1, n=_N_IN)

    Yup = _dense_gemm(h1, Wup, ad1, normed=True, n=_N_IN)
    h2 = _gather_scatter_add(Yup, up_src, up_dst, _pad_rows(skip, n_out_pad))

    Y2 = _dense_gemm(h2, W2, id_ad, normed=False, n=_N_OUT)
    h3 = _gather_scatter_add(Y2, conv2_src, conv2_dst, zero_out)
    ad2 = _lrelu_bn_stats(h3, g2, b2, n=_N_OUT)

    Y3 = _dense_gemm(h3, W3, ad2, normed=True, n=_N_OUT)
    h4 = _gather_scatter_add(Y3, conv3_src, conv3_dst, zero_out)
    ad3 = _lrelu_bn_stats(h4, g3, b3, n=_N_OUT)

    Y4 = _dense_gemm(h4, W4, ad3, normed=True, n=_N_OUT)
    h5 = _gather_scatter_add(Y4, conv4_src, conv4_dst, zero_out)
    ad4 = _lrelu_bn_stats(h5, g4, b4, n=_N_OUT)

    return _apply_norm(h5, ad4, n=_N_OUT)


# speedup vs baseline: 2.2323x; 1.0145x over previous
import dataclasses
import functools

import jax
import jax.numpy as jnp
from jax import lax
from jax.experimental import pallas as pl
from jax.experimental.pallas import tpu as pltpu
from jax.experimental.pallas import tpu_sc as plsc

_N_IN = 50000
_N_OUT = 100000
_C = 128
_EPS = 1e-5
_SLOPE = 0.01
_BN = 10000


def _stats_body(h_ref, g_ref, b_ref, o_ref, acc_ref, *, nrows, nblocks):
    i = pl.program_id(0)

    @pl.when(i == 0)
    def _():
        acc_ref[...] = jnp.zeros_like(acc_ref)

    h = h_ref[...]
    h = jnp.where(h >= 0, h, _SLOPE * h)
    acc_ref[0, :] += jnp.sum(h, axis=0)
    acc_ref[1, :] += jnp.sum(h * h, axis=0)

    @pl.when(i == nblocks - 1)
    def _():
        m = acc_ref[0, :] / nrows
        v = acc_ref[1, :] / nrows - m * m
        a = g_ref[0, :] * jax.lax.rsqrt(v + _EPS)
        o_ref[0, :] = a
        o_ref[1, :] = b_ref[0, :] - m * a


def _lrelu_bn_stats(h, g, b, n):
    nblocks = n // _BN
    return pl.pallas_call(
        functools.partial(_stats_body, nrows=float(n), nblocks=nblocks),
        grid=(nblocks,),
        in_specs=[
            pl.BlockSpec((_BN, _C), lambda i: (i, 0)),
            pl.BlockSpec((1, _C), lambda i: (0, 0)),
            pl.BlockSpec((1, _C), lambda i: (0, 0)),
        ],
        out_specs=pl.BlockSpec((2, _C), lambda i: (0, 0)),
        out_shape=jax.ShapeDtypeStruct((2, _C), jnp.float32),
        scratch_shapes=[pltpu.VMEM((2, _C), jnp.float32)],
    )(h, g.reshape(1, _C), b.reshape(1, _C))


def _gemm_body(feat_ref, w_ref, ad_ref, y_ref, *, normed):
    k = pl.program_id(1)
    f = feat_ref[...]
    wk = w_ref[k]
    hi = jax.lax.Precision.DEFAULT
    if normed:
        a = ad_ref[0, :]
        d = ad_ref[1, :]
        f = jnp.where(f >= 0, f, _SLOPE * f)
        wk = a[:, None] * wk
        t = jnp.dot(d[None, :], w_ref[k], precision=hi,
                    preferred_element_type=jnp.float32)
        y_ref[0] = jnp.dot(f, wk, precision=hi,
                           preferred_element_type=jnp.float32) + t
    else:
        y_ref[0] = jnp.dot(f, wk, precision=hi,
                           preferred_element_type=jnp.float32)


def _dense_gemm(feat, W, ad, normed, n):
    K = W.shape[0]
    nblocks = n // _BN
    return pl.pallas_call(
        functools.partial(_gemm_body, normed=normed),
        grid=(nblocks, K),
        in_specs=[
            pl.BlockSpec((_BN, _C), lambda i, k: (i, 0)),
            pl.BlockSpec((K, _C, _C), lambda i, k: (0, 0, 0)),
            pl.BlockSpec((2, _C), lambda i, k: (0, 0)),
        ],
        out_specs=pl.BlockSpec((1, _BN, _C), lambda i, k: (k, i, 0)),
        out_shape=jax.ShapeDtypeStruct((K, n, _C), jnp.float32),
    )(feat, W, ad)


def _apply_body(h_ref, ad_ref, o_ref):
    h = h_ref[...]
    h = jnp.where(h >= 0, h, _SLOPE * h)
    o_ref[...] = ad_ref[0, :] * h + ad_ref[1, :]


def _apply_norm(h, ad, n):
    return pl.pallas_call(
        _apply_body,
        grid=(n // _BN,),
        in_specs=[
            pl.BlockSpec((_BN, _C), lambda i: (i, 0)),
            pl.BlockSpec((2, _C), lambda i: (0, 0)),
        ],
        out_specs=pl.BlockSpec((_BN, _C), lambda i: (i, 0)),
        out_shape=jax.ShapeDtypeStruct((n, _C), jnp.float32),
    )(h, ad)


def _idx_body(src_ref, o_ref, *, n):
    o_ref[...] = src_ref[...] + pl.program_id(0) * n


def _flat_gather_idx(src, n):
    K, P = src.shape
    out = pl.pallas_call(
        functools.partial(_idx_body, n=n),
        grid=(K,),
        in_specs=[pl.BlockSpec((1, 1, P), lambda k: (k, 0, 0))],
        out_specs=pl.BlockSpec((1, 1, P), lambda k: (k, 0, 0)),
        out_shape=jax.ShapeDtypeStruct((K, 1, P), jnp.int32),
    )(src.reshape(K, 1, P))
    return out.reshape(K * P)


_CH = 11264
_Q = 256
_BL = 2048
_NSUB = 16


def _sc_scatter_body(y_hbm, g_hbm, d_hbm, init_hbm, out_hbm,
                     dvec, gvec, cga, cla, stage, shacc, sem, sem2,
                     *, n_pad, per_tile):
    core = lax.axis_index("c")
    sub = lax.axis_index("s")
    nchunks = n_pad // _CH
    rpt = _CH // _NSUB
    iota = lax.iota(jnp.int32, 16)

    def flush():
        ng = _Q // 128
        gath = [pltpu.async_copy(y_hbm.at[cga.at[j]],
                                 stage.at[pl.ds(j * 128, 128)], sem)
                for j in range(ng)]
        scat = []
        for j in range(ng):
            gath[j].wait()
            scat.append(pltpu.async_copy(stage.at[pl.ds(j * 128, 128)],
                                         shacc.at[cla.at[j]], sem2,
                                         add=True))
        for s in scat:
            s.wait()

    for q in range(nchunks // 2):
        chunk = 2 * q + core
        base = chunk * _CH
        pltpu.sync_copy(init_hbm.at[pl.ds(base + sub * rpt, rpt)],
                        shacc.at[pl.ds(sub * rpt, rpt)])
        plsc.subcore_barrier()

        def scan_buf(bd, bg, fill):
            def quad(j, fill):
                for u in range(8):
                    i = j * 8 + u
                    d = bd[pl.ds(i * 16, 16)]
                    g = bg[pl.ds(i * 16, 16)]
                    local = d - base
                    m = local.astype(jnp.uint32) < jnp.uint32(_CH)
                    c = plsc.cumsum(m.astype(jnp.int32))
                    pos = fill + c - 1
                    plsc.store_scatter(cga, [pos >> 7, pos & 127], g,
                                       mask=m)
                    plsc.store_scatter(cla, [pos >> 7, pos & 127], local,
                                       mask=m)
                    fill = fill + c[15]

                def do_flush():
                    flush()
                    for t in range(8):
                        s = pl.ds(t * 16, 16)
                        cga[0, s] = cga[_Q // 128, s]
                        cla[0, s] = cla[_Q // 128, s]

                pl.when(fill >= _Q)(do_flush)
                return jnp.where(fill >= _Q, fill - _Q, fill)

            return lax.fori_loop(0, _BL // 128, quad, fill)

        def blk(b, fill):
            off = sub * per_tile + b * _BL
            h1 = pltpu.async_copy(d_hbm.at[pl.ds(off, _BL)], dvec, sem2)
            h2 = pltpu.async_copy(g_hbm.at[pl.ds(off, _BL)], gvec, sem2)
            h1.wait()
            h2.wait()
            return scan_buf(dvec, gvec, fill)

        fill = lax.fori_loop(0, per_tile // _BL, blk, jnp.int32(0))

        zero16 = jnp.zeros((16,), jnp.int32)
        dump16 = jnp.full((16,), _CH, jnp.int32)
        for j in range(_Q // 16):
            posj = fill + j * 16 + iota
            mj = posj < _Q + 16
            plsc.store_scatter(cga, [posj >> 7, posj & 127], zero16, mask=mj)
            plsc.store_scatter(cla, [posj >> 7, posj & 127], dump16, mask=mj)
        flush()
        plsc.subcore_barrier()
        pltpu.sync_copy(shacc.at[pl.ds(sub * rpt, rpt)],
                        out_hbm.at[pl.ds(base + sub * rpt, rpt)])
        plsc.subcore_barrier()


def _sc_gather_scatter_add(Y2d, gidx, dst, init):
    n_pad = init.shape[0]
    kp_pad = gidx.shape[0]
    per_tile = kp_pad // _NSUB
    mesh = plsc.VectorSubcoreMesh(core_axis_name="c", subcore_axis_name="s")
    cp = pltpu.CompilerParams()
    if "needs_layout_passes" in pltpu.CompilerParams.__dataclass_fields__:
        cp = dataclasses.replace(cp, needs_layout_passes=False)
    kern = pl.kernel(
        functools.partial(_sc_scatter_body, n_pad=n_pad, per_tile=per_tile),
        mesh=mesh,
        out_type=jax.ShapeDtypeStruct((n_pad, _C), jnp.float32),
        scratch_types=[
            pltpu.VMEM((_BL,), jnp.int32),
            pltpu.VMEM((_BL,), jnp.int32),
            pltpu.VMEM((_Q // 128 + 1, 128), jnp.int32),
            pltpu.VMEM((_Q // 128 + 1, 128), jnp.int32),
            pltpu.VMEM((_Q, _C), jnp.float32),
            pltpu.VMEM_SHARED((_CH + 8, _C), jnp.float32),
            pltpu.SemaphoreType.DMA,
            pltpu.SemaphoreType.DMA,
        ],
        compiler_params=cp,
    )
    return kern(Y2d, gidx, dst, init)


def _pad_pairs(idx, pad_to, fill):
    flat = idx.reshape(-1)
    return jnp.pad(flat, (0, pad_to - flat.shape[0]), constant_values=fill)


def _gather_scatter_add(Y, src, dst, init):
    K, n, Cc = Y.shape
    kp = K * src.shape[1]
    kp_pad = -(-kp // (_NSUB * _BL)) * (_NSUB * _BL)
    gidx = _pad_pairs(_flat_gather_idx(src, n), kp_pad, 0)
    dstf = _pad_pairs(dst, kp_pad, -(2 ** 30))
    return _sc_gather_scatter_add(Y.reshape(K * n, Cc), gidx, dstf, init)


_ID_AD = None


def _pad_rows(a, n_pad):
    return jnp.pad(a, ((0, n_pad - a.shape[0]), (0, 0)))


def kernel(x, skip, W1, Wup, W2, W3, W4, g1, b1, g2, b2, g3, b3, g4, b4,
           conv1_src, conv1_dst, up_src, up_dst, conv2_src, conv2_dst,
           conv3_src, conv3_dst, conv4_src, conv4_dst):
    n_in_pad = -(-_N_IN // (2 * _CH)) * (2 * _CH)
    n_out_pad = -(-_N_OUT // (2 * _CH)) * (2 * _CH)
    zero_in = jnp.zeros((n_in_pad, _C), jnp.float32)
    zero_out = jnp.zeros((n_out_pad, _C), jnp.float32)
    id_ad = jnp.zeros((2, _C), jnp.float32)

    Y1 = _dense_gemm(x, W1, id_ad, normed=False, n=_N_IN)
    h1 = _gather_scatter_add(Y1, conv1_src, conv1_dst, zero_in)
    ad1 = _lrelu_bn_stats(h1, g1, b1, n=_N_IN)

    Yup = _dense_gemm(h1, Wup, ad1, normed=True, n=_N_IN)
    h2 = _gather_scatter_add(Yup, up_src, up_dst, _pad_rows(skip, n_out_pad))

    Y2 = _dense_gemm(h2, W2, id_ad, normed=False, n=_N_OUT)
    h3 = _gather_scatter_add(Y2, conv2_src, conv2_dst, zero_out)
    ad2 = _lrelu_bn_stats(h3, g2, b2, n=_N_OUT)

    Y3 = _dense_gemm(h3, W3, ad2, normed=True, n=_N_OUT)
    h4 = _gather_scatter_add(Y3, conv3_src, conv3_dst, zero_out)
    ad3 = _lrelu_bn_stats(h4, g3, b3, n=_N_OUT)

    Y4 = _dense_gemm(h4, W4, ad3, normed=True, n=_N_OUT)
    h5 = _gather_scatter_add(Y4, conv4_src, conv4_dst, zero_out)
    ad4 = _lrelu_bn_stats(h5, g4, b4, n=_N_OUT)

    return _apply_norm(h5, ad4, n=_N_OUT)


# speedup vs baseline: 3.3117x; 1.4835x over previous
import dataclasses
import functools

import jax
import jax.numpy as jnp
from jax import lax
from jax.experimental import pallas as pl
from jax.experimental.pallas import tpu as pltpu
from jax.experimental.pallas import tpu_sc as plsc

_N_IN = 50000
_N_OUT = 100000
_C = 128
_EPS = 1e-5
_SLOPE = 0.01
_BN = 10000


def _stats_body(h_ref, g_ref, b_ref, o_ref, acc_ref, *, nrows, nblocks):
    i = pl.program_id(0)

    @pl.when(i == 0)
    def _():
        acc_ref[...] = jnp.zeros_like(acc_ref)

    h = h_ref[...]
    h = jnp.where(h >= 0, h, _SLOPE * h)
    acc_ref[0, :] += jnp.sum(h, axis=0)
    acc_ref[1, :] += jnp.sum(h * h, axis=0)

    @pl.when(i == nblocks - 1)
    def _():
        m = acc_ref[0, :] / nrows
        v = acc_ref[1, :] / nrows - m * m
        a = g_ref[0, :] * jax.lax.rsqrt(v + _EPS)
        o_ref[0, :] = a
        o_ref[1, :] = b_ref[0, :] - m * a


def _lrelu_bn_stats(h, g, b, n):
    nblocks = n // _BN
    return pl.pallas_call(
        functools.partial(_stats_body, nrows=float(n), nblocks=nblocks),
        grid=(nblocks,),
        in_specs=[
            pl.BlockSpec((_BN, _C), lambda i: (i, 0)),
            pl.BlockSpec((1, _C), lambda i: (0, 0)),
            pl.BlockSpec((1, _C), lambda i: (0, 0)),
        ],
        out_specs=pl.BlockSpec((2, _C), lambda i: (0, 0)),
        out_shape=jax.ShapeDtypeStruct((2, _C), jnp.float32),
        scratch_shapes=[pltpu.VMEM((2, _C), jnp.float32)],
    )(h, g.reshape(1, _C), b.reshape(1, _C))


def _gemm_body(feat_ref, w_ref, ad_ref, y_ref, *, normed):
    k = pl.program_id(1)
    f = feat_ref[...]
    wk = w_ref[k]
    hi = jax.lax.Precision.DEFAULT
    if normed:
        a = ad_ref[0, :]
        d = ad_ref[1, :]
        f = jnp.where(f >= 0, f, _SLOPE * f)
        wk = a[:, None] * wk
        t = jnp.dot(d[None, :], w_ref[k], precision=hi,
                    preferred_element_type=jnp.float32)
        y_ref[0] = jnp.dot(f, wk, precision=hi,
                           preferred_element_type=jnp.float32) + t
    else:
        y_ref[0] = jnp.dot(f, wk, precision=hi,
                           preferred_element_type=jnp.float32)


def _dense_gemm(feat, W, ad, normed, n):
    K = W.shape[0]
    nblocks = n // _BN
    return pl.pallas_call(
        functools.partial(_gemm_body, normed=normed),
        grid=(nblocks, K),
        in_specs=[
            pl.BlockSpec((_BN, _C), lambda i, k: (i, 0)),
            pl.BlockSpec((K, _C, _C), lambda i, k: (0, 0, 0)),
            pl.BlockSpec((2, _C), lambda i, k: (0, 0)),
        ],
        out_specs=pl.BlockSpec((1, _BN, _C), lambda i, k: (k, i, 0)),
        out_shape=jax.ShapeDtypeStruct((K, n, _C), jnp.float32),
    )(feat, W, ad)


def _apply_body(h_ref, ad_ref, o_ref):
    h = h_ref[...]
    h = jnp.where(h >= 0, h, _SLOPE * h)
    o_ref[...] = ad_ref[0, :] * h + ad_ref[1, :]


def _apply_norm(h, ad, n):
    return pl.pallas_call(
        _apply_body,
        grid=(n // _BN,),
        in_specs=[
            pl.BlockSpec((_BN, _C), lambda i: (i, 0)),
            pl.BlockSpec((2, _C), lambda i: (0, 0)),
        ],
        out_specs=pl.BlockSpec((_BN, _C), lambda i: (i, 0)),
        out_shape=jax.ShapeDtypeStruct((n, _C), jnp.float32),
    )(h, ad)


def _idx_body(src_ref, o_ref, *, n):
    o_ref[...] = src_ref[...] + pl.program_id(0) * n


def _flat_gather_idx(src, n):
    K, P = src.shape
    out = pl.pallas_call(
        functools.partial(_idx_body, n=n),
        grid=(K,),
        in_specs=[pl.BlockSpec((1, 1, P), lambda k: (k, 0, 0))],
        out_specs=pl.BlockSpec((1, 1, P), lambda k: (k, 0, 0)),
        out_shape=jax.ShapeDtypeStruct((K, 1, P), jnp.int32),
    )(src.reshape(K, 1, P))
    return out.reshape(K * P)


_CH = 13312
_Q = 128
_BL = 2048
_NSUB = 16


def _sc_scatter_body(y_hbm, g_hbm, d_hbm, init_hbm, out_hbm,
                     dvec, gvec, cga, cla, stage, shacc, sem, sem2,
                     *, n_pad, per_tile):
    core = lax.axis_index("c")
    sub = lax.axis_index("s")
    nchunks = n_pad // _CH
    rpt = _CH // _NSUB
    iota = lax.iota(jnp.int32, 16)

    def flush():
        ng = _Q // 128
        gath = [pltpu.async_copy(y_hbm.at[cga.at[j]],
                                 stage.at[pl.ds(j * 128, 128)], sem)
                for j in range(ng)]
        scat = []
        for j in range(ng):
            gath[j].wait()
            scat.append(pltpu.async_copy(stage.at[pl.ds(j * 128, 128)],
                                         shacc.at[cla.at[j]], sem2,
                                         add=True))
        for s in scat:
            s.wait()

    for q in range(nchunks // 2):
        chunk = 2 * q + core
        base = chunk * _CH
        pltpu.sync_copy(init_hbm.at[pl.ds(base + sub * rpt, rpt)],
                        shacc.at[pl.ds(sub * rpt, rpt)])
        plsc.subcore_barrier()

        def scan_buf(bd, bg, fill):
            def quad(j, fill):
                for u in range(8):
                    i = j * 8 + u
                    d = bd[pl.ds(i * 16, 16)]
                    g = bg[pl.ds(i * 16, 16)]
                    local = d - base
                    m = local.astype(jnp.uint32) < jnp.uint32(_CH)
                    c = plsc.cumsum(m.astype(jnp.int32))
                    pos = fill + c - 1
                    plsc.store_scatter(cga, [pos >> 7, pos & 127], g,
                                       mask=m)
                    plsc.store_scatter(cla, [pos >> 7, pos & 127], local,
                                       mask=m)
                    fill = fill + c[15]

                def do_flush():
                    flush()
                    for t in range(8):
                        s = pl.ds(t * 16, 16)
                        cga[0, s] = cga[_Q // 128, s]
                        cla[0, s] = cla[_Q // 128, s]

                pl.when(fill >= _Q)(do_flush)
                return jnp.where(fill >= _Q, fill - _Q, fill)

            return lax.fori_loop(0, _BL // 128, quad, fill)

        def blk(b, fill):
            off = sub * per_tile + b * _BL
            h1 = pltpu.async_copy(d_hbm.at[pl.ds(off, _BL)], dvec, sem2)
            h2 = pltpu.async_copy(g_hbm.at[pl.ds(off, _BL)], gvec, sem2)
            h1.wait()
            h2.wait()
            return scan_buf(dvec, gvec, fill)

        fill = lax.fori_loop(0, per_tile // _BL, blk, jnp.int32(0))

        zero16 = jnp.zeros((16,), jnp.int32)
        dump16 = jnp.full((16,), _CH, jnp.int32)
        for j in range(_Q // 16):
            posj = fill + j * 16 + iota
            mj = posj < _Q + 16
            plsc.store_scatter(cga, [posj >> 7, posj & 127], zero16, mask=mj)
            plsc.store_scatter(cla, [posj >> 7, posj & 127], dump16, mask=mj)
        flush()
        plsc.subcore_barrier()
        pltpu.sync_copy(shacc.at[pl.ds(sub * rpt, rpt)],
                        out_hbm.at[pl.ds(base + sub * rpt, rpt)])
        plsc.subcore_barrier()


def _sc_gather_scatter_add(Y2d, gidx, dst, init):
    n_pad = init.shape[0]
    kp_pad = gidx.shape[0]
    per_tile = kp_pad // _NSUB
    mesh = plsc.VectorSubcoreMesh(core_axis_name="c", subcore_axis_name="s")
    cp = pltpu.CompilerParams()
    if "needs_layout_passes" in pltpu.CompilerParams.__dataclass_fields__:
        cp = dataclasses.replace(cp, needs_layout_passes=False)
    kern = pl.kernel(
        functools.partial(_sc_scatter_body, n_pad=n_pad, per_tile=per_tile),
        mesh=mesh,
        out_type=jax.ShapeDtypeStruct((n_pad, _C), jnp.float32),
        scratch_types=[
            pltpu.VMEM((_BL,), jnp.int32),
            pltpu.VMEM((_BL,), jnp.int32),
            pltpu.VMEM((_Q // 128 + 1, 128), jnp.int32),
            pltpu.VMEM((_Q // 128 + 1, 128), jnp.int32),
            pltpu.VMEM((_Q, _C), jnp.float32),
            pltpu.VMEM_SHARED((_CH + 8, _C), jnp.float32),
            pltpu.SemaphoreType.DMA,
            pltpu.SemaphoreType.DMA,
        ],
        compiler_params=cp,
    )
    return kern(Y2d, gidx, dst, init)


def _pad_pairs(idx, pad_to, fill):
    flat = idx.reshape(-1)
    return jnp.pad(flat, (0, pad_to - flat.shape[0]), constant_values=fill)


def _gather_scatter_add(Y, src, dst, init):
    K, n, Cc = Y.shape
    kp = K * src.shape[1]
    kp_pad = -(-kp // (_NSUB * _BL)) * (_NSUB * _BL)
    gidx = _pad_pairs(_flat_gather_idx(src, n), kp_pad, 0)
    dstf = _pad_pairs(dst, kp_pad, -(2 ** 30))
    return _sc_gather_scatter_add(Y.reshape(K * n, Cc), gidx, dstf, init)


_ID_AD = None


def _pad_rows(a, n_pad):
    return jnp.pad(a, ((0, n_pad - a.shape[0]), (0, 0)))


def kernel(x, skip, W1, Wup, W2, W3, W4, g1, b1, g2, b2, g3, b3, g4, b4,
           conv1_src, conv1_dst, up_src, up_dst, conv2_src, conv2_dst,
           conv3_src, conv3_dst, conv4_src, conv4_dst):
    n_in_pad = -(-_N_IN // (2 * _CH)) * (2 * _CH)
    n_out_pad = -(-_N_OUT // (2 * _CH)) * (2 * _CH)
    zero_in = jnp.zeros((n_in_pad, _C), jnp.float32)
    zero_out = jnp.zeros((n_out_pad, _C), jnp.float32)
    id_ad = jnp.zeros((2, _C), jnp.float32)

    Y1 = _dense_gemm(x, W1, id_ad, normed=False, n=_N_IN)
    h1 = _gather_scatter_add(Y1, conv1_src, conv1_dst, zero_in)
    ad1 = _lrelu_bn_stats(h1, g1, b1, n=_N_IN)

    Yup = _dense_gemm(h1, Wup, ad1, normed=True, n=_N_IN)
    h2 = _gather_scatter_add(Yup, up_src, up_dst, _pad_rows(skip, n_out_pad))

    Y2 = _dense_gemm(h2, W2, id_ad, normed=False, n=_N_OUT)
    h3 = _gather_scatter_add(Y2, conv2_src, conv2_dst, zero_out)
    ad2 = _lrelu_bn_stats(h3, g2, b2, n=_N_OUT)

    Y3 = _dense_gemm(h3, W3, ad2, normed=True, n=_N_OUT)
    h4 = _gather_scatter_add(Y3, conv3_src, conv3_dst, zero_out)
    ad3 = _lrelu_bn_stats(h4, g3, b3, n=_N_OUT)

    Y4 = _dense_gemm(h4, W4, ad3, normed=True, n=_N_OUT)
    h5 = _gather_scatter_add(Y4, conv4_src, conv4_dst, zero_out)
    ad4 = _lrelu_bn_stats(h5, g4, b4, n=_N_OUT)

    return _apply_norm(h5, ad4, n=_N_OUT)
